# baseline (speedup 1.0000x reference)
import functools

import jax
import jax.numpy as jnp
from jax.experimental import pallas as pl
from jax.experimental.pallas import tpu as pltpu

N_ATOM = 32
DIM = 64
N_LAYER = 4
OFFS = (1, 2, -1, -2)


def _leaky(v):
    return jnp.maximum(v, 0.01 * v)


def _mm(a, w):
    return jax.lax.dot_general(a, w, (((1,), (0,)), ((), ())),
                               precision=jax.lax.Precision.HIGHEST,
                               preferred_element_type=jnp.float32)


def _roll(t, off, q):
    m = t.shape[1]
    k = (off * q) % m
    if k == 0:
        return t
    return jnp.concatenate([t[:, k:], t[:, :k]], axis=1)


def _egnn_block(x_ref, f0w, f0b, eW1a, eW1b, ew1c, eb1, eW2, eb2, cw, cb,
                nW1h, nW1g, nb1, nW2, nb2, pw, pb, out_ref, *, q):
    cset = x_ref[:]
    h = _leaky(_mm(f0w[:], cset) + f0b[:])
    for l in range(N_LAYER):
        ha = _mm(eW1a[l], h) + eb1[l]
        hb = _mm(eW1b[l], h)
        dts = {}
        for off in (1, 2):
            rel = cset - _roll(cset, off, q)
            dts[off] = _mm(ew1c[l], rel * rel)
        dts[-1] = _roll(dts[1], -1, q)
        dts[-2] = _roll(dts[2], -2, q)
        aggr = jnp.zeros_like(h)
        for off in OFFS:
            spre = _roll(ha, -off, q) + hb + dts[-off]
            aggr = aggr + _leaky(_mm(eW2[l], _leaky(spre)) + eb2[l])
        z = jnp.sum(aggr * cw[l], axis=0, keepdims=True)
        cu = jnp.tanh(z + cb[l])
        cset = cset + cu * 0.1
        u = _leaky(_mm(nW1h[l], h) + _mm(nW1g[l], aggr) + nb1[l])
        h = h + _leaky(_mm(nW2[l], u) + nb2[l])
    s = h
    w = s.shape[1]
    while w > q:
        w //= 2
        s = s[:, :w] + s[:, w:2 * w]
    out_ref[:] = _leaky(_mm(pw[:], s) + pb[:])


@jax.jit
def kernel(x, f0_W, f0_b, eW1, eb1, eW2, eb2, cW, cb, nW1, nb1, nW2, nb2,
           pW, pb, edge_index):
    del edge_index
    B = x.shape[0]
    q = 256
    grid = (B // q,)
    G = B // q

    xt = x.reshape(G, q, N_ATOM, 3).transpose(3, 0, 2, 1).reshape(3, B * N_ATOM)

    tT = lambda w: jnp.swapaxes(w, 1, 2)
    f0wT = f0_W.T
    f0bc = f0_b[:, None]
    eW1aT = tT(eW1[:, :DIM, :])
    eW1bT = tT(eW1[:, DIM:2 * DIM, :])
    ew1cT = jnp.repeat(eW1[:, 2 * DIM, :][:, :, None], 3, axis=2)
    eb1c = eb1[:, :, None]
    eW2T = tT(eW2)
    eb2c = eb2[:, :, None]
    nW1hT = tT(nW1[:, :DIM, :])
    nW1gT = tT(nW1[:, DIM:, :]) * 0.25
    nb1c = nb1[:, :, None]
    nW2T = tT(nW2)
    nb2c = nb2[:, :, None]
    cwc = (cW[:, :, 0] * 0.25)[:, :, None]
    cbc = cb[:, :, None]
    pwT = pW[:, 0][None, :] / N_ATOM
    pbc = pb[None, :]

    rep = lambda shape: pl.BlockSpec(shape, lambda i: (0,) * len(shape))
    out = pl.pallas_call(
        functools.partial(_egnn_block, q=q),
        grid=grid,
        in_specs=[
            pl.BlockSpec((3, N_ATOM * q), lambda i: (0, i)),
            rep(f0wT.shape), rep(f0bc.shape),
            rep(eW1aT.shape), rep(eW1bT.shape), rep(ew1cT.shape),
            rep(eb1c.shape),
            rep(eW2T.shape), rep(eb2c.shape),
            rep(cwc.shape), rep(cbc.shape),
            rep(nW1hT.shape), rep(nW1gT.shape), rep(nb1c.shape),
            rep(nW2T.shape), rep(nb2c.shape),
            rep(pwT.shape), rep(pbc.shape),
        ],
        out_specs=pl.BlockSpec((1, q), lambda i: (0, i)),
        out_shape=jax.ShapeDtypeStruct((1, B), jnp.float32),
        compiler_params=pltpu.CompilerParams(
            dimension_semantics=("parallel",)),
    )(xt, f0wT, f0bc, eW1aT, eW1bT, ew1cT, eb1c, eW2T, eb2c, cwc, cbc,
      nW1hT, nW1gT, nb1c, nW2T, nb2c, pwT, pbc)
    return out.reshape(B, 1)


# speedup vs baseline: 3.4605x; 3.4605x over previous
import functools

import jax
import jax.numpy as jnp
from jax.experimental import pallas as pl
from jax.experimental.pallas import tpu as pltpu

N_ATOM = 32
DIM = 64
N_LAYER = 4
OFFS = (1, 2, -1, -2)


def _leaky(v):
    return jnp.maximum(v, 0.01 * v)


def _mm(a, w):
    return jax.lax.dot_general(a, w, (((1,), (0,)), ((), ())),
                               preferred_element_type=jnp.float32)


def _roll(t, off, q):
    m = t.shape[1]
    k = (off * q) % m
    if k == 0:
        return t
    return jnp.concatenate([t[:, k:], t[:, :k]], axis=1)


def _egnn_block(x_ref, f0w, f0b, eW1a, eW1b, ew1c, eb1, eW2, eb2, cw, cb,
                nW1h, nW1g, nb1, nW2, nb2, pw, pb, out_ref, *, q):
    cset = x_ref[:]
    h = _leaky(_mm(f0w[:], cset) + f0b[:])
    for l in range(N_LAYER):
        ha = _mm(eW1a[l], h) + eb1[l]
        hb = _mm(eW1b[l], h)
        dts = {}
        for off in (1, 2):
            rel = cset - _roll(cset, off, q)
            dsq = jnp.sum(rel * rel, axis=0, keepdims=True)
            dts[off] = _mm(ew1c[l], dsq)
        dts[-1] = _roll(dts[1], -1, q)
        dts[-2] = _roll(dts[2], -2, q)
        aggr = jnp.zeros_like(h)
        for off in OFFS:
            spre = _roll(ha, -off, q) + hb + dts[-off]
            aggr = aggr + _leaky(_mm(eW2[l], _leaky(spre)) + eb2[l])
        z = _mm(cw[l], aggr)
        cu = jnp.tanh(z + cb[l])
        cset = cset + cu * 0.1
        u = _leaky(_mm(nW1h[l], h) + _mm(nW1g[l], aggr) + nb1[l])
        h = h + _leaky(_mm(nW2[l], u) + nb2[l])
    s = h
    w = s.shape[1]
    while w > q:
        w //= 2
        s = s[:, :w] + s[:, w:2 * w]
    out_ref[:] = _leaky(_mm(pw[:], s) + pb[:])


@jax.jit
def kernel(x, f0_W, f0_b, eW1, eb1, eW2, eb2, cW, cb, nW1, nb1, nW2, nb2,
           pW, pb, edge_index):
    del edge_index
    B = x.shape[0]
    q = 256
    grid = (B // q,)
    G = B // q

    xt = x.reshape(G, q, N_ATOM, 3).transpose(3, 0, 2, 1).reshape(3, B * N_ATOM)

    tT = lambda w: jnp.swapaxes(w, 1, 2)
    f0wT = f0_W.T
    f0bc = f0_b[:, None]
    eW1aT = tT(eW1[:, :DIM, :])
    eW1bT = tT(eW1[:, DIM:2 * DIM, :])
    ew1cT = eW1[:, 2 * DIM, :][:, :, None]
    eb1c = eb1[:, :, None]
    eW2T = tT(eW2)
    eb2c = eb2[:, :, None]
    nW1hT = tT(nW1[:, :DIM, :])
    nW1gT = tT(nW1[:, DIM:, :]) * 0.25
    nb1c = nb1[:, :, None]
    nW2T = tT(nW2)
    nb2c = nb2[:, :, None]
    cwc = (cW[:, :, 0] * 0.25)[:, None, :]
    cbc = cb[:, :, None]
    pwT = pW[:, 0][None, :] / N_ATOM
    pbc = pb[None, :]

    rep = lambda shape: pl.BlockSpec(shape, lambda i: (0,) * len(shape))
    out = pl.pallas_call(
        functools.partial(_egnn_block, q=q),
        grid=grid,
        in_specs=[
            pl.BlockSpec((3, N_ATOM * q), lambda i: (0, i)),
            rep(f0wT.shape), rep(f0bc.shape),
            rep(eW1aT.shape), rep(eW1bT.shape), rep(ew1cT.shape),
            rep(eb1c.shape),
            rep(eW2T.shape), rep(eb2c.shape),
            rep(cwc.shape), rep(cbc.shape),
            rep(nW1hT.shape), rep(nW1gT.shape), rep(nb1c.shape),
            rep(nW2T.shape), rep(nb2c.shape),
            rep(pwT.shape), rep(pbc.shape),
        ],
        out_specs=pl.BlockSpec((1, q), lambda i: (0, i)),
        out_shape=jax.ShapeDtypeStruct((1, B), jnp.float32),
        compiler_params=pltpu.CompilerParams(
            dimension_semantics=("parallel",)),
    )(xt, f0wT, f0bc, eW1aT, eW1bT, ew1cT, eb1c, eW2T, eb2c, cwc, cbc,
      nW1hT, nW1gT, nb1c, nW2T, nb2c, pwT, pbc)
    return out.reshape(B, 1)
